# baseline (speedup 1.0000x reference)
import jax
import jax.numpy as jnp
from jax.experimental import pallas as pl
from jax.experimental.pallas import tpu as pltpu

_MEL = 80
_HID = 384
_K = 5
_STRIDE = 8
_NB = 6
_VQB = 1024
_VQD = 256
_B = 16
_T = 2048
_T2 = _T // _STRIDE

_PREC = jax.lax.Precision.HIGHEST


def _mm(a, b):
    return jax.lax.dot_general(a, b, (((1,), (0,)), ((), ())),
                               precision=_PREC,
                               preferred_element_type=jnp.float32)


def _conv(x, Wt, b):
    T = x.shape[0]
    xp = jnp.pad(x, ((_K // 2, _K // 2), (0, 0)))
    out = None
    for k in range(_K):
        part = _mm(xp[k:k + T], Wt[k])
        out = part if out is None else out + part
    return out + b


def _mmT(w, x):
    return jax.lax.dot_general(w, x, (((1,), (0,)), ((), ())),
                               precision=_PREC,
                               preferred_element_type=jnp.float32)


def _convT(x, Wt, b):
    T = x.shape[1]
    xp = jnp.pad(x, ((0, 0), (_K // 2, _K // 2)))
    out = None
    for k in range(_K):
        part = _mmT(Wt[k], xp[:, k:k + T])
        out = part if out is None else out + part
    return out + b


def _stage1_body(mel_ref, Win_ref, bin_ref, Wpre_ref, bpre_ref, hmid_ref):
    x = mel_ref[0]
    h = jnp.maximum(_convT(x, Win_ref[...], bin_ref[...]), 0.0)
    for i in range(_NB):
        h = h + jnp.maximum(_convT(h, Wpre_ref[i], bpre_ref[i]), 0.0)
    hmid_ref[0] = jnp.max(h.reshape(_HID, _T2, _STRIDE), axis=2)


_PB = _T2 + 8
_S2B = 4
_ROWS = _B // _S2B * _PB
_CHUNK = 2 * _PB


def _stage2_body(hpad_ref, Wpost_ref, bpost_ref, Wout_ref, bout_ref, cb_ref,
                 zq_ref, loss_ref):
    bidx = pl.program_id(0)
    h = hpad_ref[0]

    r = jax.lax.broadcasted_iota(jnp.int32, (_ROWS, 1), 0) % _PB
    rm = jnp.logical_and(r >= 4, r < 4 + _T2).astype(jnp.float32)

    for i in range(_NB):
        h = rm * (h + jnp.maximum(_conv(h, Wpost_ref[i], bpost_ref[i:i + 1]),
                                  0.0))

    ze = _conv(h, Wout_ref[...], bout_ref[...])

    cb = cb_ref[...]
    cbn = 0.5 * jnp.sum(cb * cb, axis=1)
    s = None
    for c in range(_ROWS // _CHUNK):
        zec = ze[c * _CHUNK:(c + 1) * _CHUNK]
        score = jax.lax.dot_general(
            zec, cb, (((1,), (1,)), ((), ())), precision=_PREC,
            preferred_element_type=jnp.float32)
        score = score - cbn[None, :]
        idx = jnp.argmax(score, axis=1)
        iota = jax.lax.broadcasted_iota(jnp.int32, (_CHUNK, _VQB), 1)
        onehot = (iota == idx[:, None]).astype(jnp.float32)
        q = _mm(onehot, cb)
        for half in range(2):
            lo = half * _PB + 4
            qv = q[lo:lo + _T2]
            zv = zec[lo:lo + _T2]
            zq_ref[0, pl.ds(c * 2 * _T2 + half * _T2, _T2), :] = qv
            e = zv - qv
            part = jnp.sum(e * e)
            s = part if s is None else s + part

    s = s[None, None]
    loss_ref[...] = jnp.where(bidx == 0, s, loss_ref[...] + s)


def kernel(mel, W_in, b_in, W_pre, b_pre, W_post, b_post, W_out, b_out,
           codebook):
    WinT = jnp.transpose(W_in, (2, 0, 1))
    WpreT = jnp.transpose(W_pre, (0, 3, 1, 2))
    Wpost_t = jnp.transpose(W_post, (0, 3, 2, 1))
    Wout_t = jnp.transpose(W_out, (2, 1, 0))
    binT = b_in[:, None]
    bpreT = b_pre[:, :, None]
    bout2 = b_out[None, :]
    melT = jnp.transpose(mel, (0, 2, 1))

    def full(a):
        return pl.BlockSpec(a.shape, lambda *g: (0,) * a.ndim)

    hmidT = pl.pallas_call(
        _stage1_body,
        grid=(_B,),
        in_specs=[
            pl.BlockSpec((1, _MEL, _T), lambda b: (b, 0, 0)),
            full(WinT), full(binT), full(WpreT), full(bpreT),
        ],
        out_specs=pl.BlockSpec((1, _HID, _T2), lambda b: (b, 0, 0)),
        out_shape=jax.ShapeDtypeStruct((_B, _HID, _T2), jnp.float32),
    )(melT, WinT, binT, WpreT, bpreT)

    hmid = jnp.transpose(hmidT, (0, 2, 1))
    hpad = jnp.pad(hmid, ((0, 0), (4, 4), (0, 0)))
    hpad = hpad.reshape(_S2B, _ROWS, _HID)

    zq, loss_sum = pl.pallas_call(
        _stage2_body,
        grid=(_S2B,),
        in_specs=[
            pl.BlockSpec((1, _ROWS, _HID), lambda b: (b, 0, 0)),
            full(Wpost_t), full(b_post), full(Wout_t),
            full(bout2), full(codebook),
        ],
        out_specs=[
            pl.BlockSpec((1, _ROWS // _PB * _T2, _VQD), lambda b: (b, 0, 0)),
            pl.BlockSpec((1, 1), lambda b: (0, 0)),
        ],
        out_shape=[
            jax.ShapeDtypeStruct((_S2B, _ROWS // _PB * _T2, _VQD),
                                 jnp.float32),
            jax.ShapeDtypeStruct((1, 1), jnp.float32),
        ],
    )(hpad, Wpost_t, b_post, Wout_t, bout2, codebook)

    zq = zq.reshape(_B, _T2, _VQD)
    loss = loss_sum[0, 0] / jnp.float32(_B * _T2 * _VQD)
    return (zq, loss, loss)


# speedup vs baseline: 1.1461x; 1.1461x over previous
import jax
import jax.numpy as jnp
from jax.experimental import pallas as pl
from jax.experimental.pallas import tpu as pltpu

_MEL = 80
_HID = 384
_K = 5
_STRIDE = 8
_NB = 6
_VQB = 1024
_VQD = 256
_B = 16
_T = 2048
_T2 = _T // _STRIDE

_PREC = jax.lax.Precision.HIGHEST


def _mm(a, b):
    return jax.lax.dot_general(a, b, (((1,), (0,)), ((), ())),
                               precision=_PREC,
                               preferred_element_type=jnp.float32)


def _conv(x, Wt, b):
    T = x.shape[0]
    xp = jnp.pad(x, ((_K // 2, _K // 2), (0, 0)))
    out = None
    for k in range(_K):
        part = _mm(xp[k:k + T], Wt[k])
        out = part if out is None else out + part
    return out + b


def _mmT(w, x):
    return jax.lax.dot_general(w, x, (((1,), (0,)), ((), ())),
                               precision=_PREC,
                               preferred_element_type=jnp.float32)


def _convT(x, Wcat, b):
    cin = x.shape[0]
    T = x.shape[1]
    xp = jnp.pad(x, ((0, 0), (_K // 2, _K // 2)))
    out = None
    for a, z in ((0, 2), (2, 4), (4, 5)):
        if z - a > 1:
            xg = jnp.concatenate([xp[:, k:k + T] for k in range(a, z)],
                                 axis=0)
        else:
            xg = xp[:, a:a + T]
        part = _mmT(Wcat[:, a * cin:z * cin], xg)
        out = part if out is None else out + part
    return out + b


def _convT5(x, Wt, b):
    T = x.shape[1]
    xp = jnp.pad(x, ((0, 0), (_K // 2, _K // 2)))
    out = None
    for k in range(_K):
        part = _mmT(Wt[k], xp[:, k:k + T])
        out = part if out is None else out + part
    return out + b


def _stage1_body(mel_ref, Win_ref, bin_ref, Wpre_ref, bpre_ref, hmid_ref):
    x = mel_ref[0]
    h = jnp.maximum(_convT5(x, Win_ref[...], bin_ref[...]), 0.0)
    for i in range(_NB):
        h = h + jnp.maximum(_convT(h, Wpre_ref[i], bpre_ref[i]), 0.0)
    hmid_ref[0] = jnp.max(h.reshape(_HID, _T2, _STRIDE), axis=2)


def _stage2_body(hmid_ref, Wpost_ref, bpost_ref, Wout_ref, bout_ref, cb_ref,
                 zq_ref, loss_ref):
    bidx = pl.program_id(0)
    h = hmid_ref[0]

    for i in range(_NB):
        h = h + jnp.maximum(_conv(h, Wpost_ref[i], bpost_ref[i:i + 1]), 0.0)

    ze = _conv(h, Wout_ref[...], bout_ref[...])

    cb = cb_ref[...]
    score = jax.lax.dot_general(
        ze, cb, (((1,), (1,)), ((), ())), precision=_PREC,
        preferred_element_type=jnp.float32)
    cbn = 0.5 * jnp.sum(cb * cb, axis=1)
    score = score - cbn[None, :]
    idx = jnp.argmax(score, axis=1)

    iota = jax.lax.broadcasted_iota(jnp.int32, (_T2, _VQB), 1)
    onehot = (iota == idx[:, None]).astype(jnp.float32)
    q = _mm(onehot, cb)

    zq_ref[0] = q
    err = ze - q
    s = jnp.sum(err * err)[None, None]
    loss_ref[...] = jnp.where(bidx == 0, s, loss_ref[...] + s)


def kernel(mel, W_in, b_in, W_pre, b_pre, W_post, b_post, W_out, b_out,
           codebook):
    WinT = jnp.transpose(W_in, (2, 0, 1))
    WpreT = jnp.transpose(W_pre, (0, 1, 3, 2)).reshape(_NB, _HID,
                                                       _K * _HID)
    Wpost_t = jnp.transpose(W_post, (0, 3, 2, 1))
    Wout_t = jnp.transpose(W_out, (2, 1, 0))
    binT = b_in[:, None]
    bpreT = b_pre[:, :, None]
    bout2 = b_out[None, :]
    melT = jnp.transpose(mel, (0, 2, 1))

    def full(a):
        return pl.BlockSpec(a.shape, lambda *g: (0,) * a.ndim)

    hmidT = pl.pallas_call(
        _stage1_body,
        grid=(_B,),
        in_specs=[
            pl.BlockSpec((1, _MEL, _T), lambda b: (b, 0, 0)),
            full(WinT), full(binT), full(WpreT), full(bpreT),
        ],
        out_specs=pl.BlockSpec((1, _HID, _T2), lambda b: (b, 0, 0)),
        out_shape=jax.ShapeDtypeStruct((_B, _HID, _T2), jnp.float32),
    )(melT, WinT, binT, WpreT, bpreT)

    hmid = jnp.transpose(hmidT, (0, 2, 1))

    zq, loss_sum = pl.pallas_call(
        _stage2_body,
        grid=(_B,),
        in_specs=[
            pl.BlockSpec((1, _T2, _HID), lambda b: (b, 0, 0)),
            full(Wpost_t), full(b_post), full(Wout_t),
            full(bout2), full(codebook),
        ],
        out_specs=[
            pl.BlockSpec((1, _T2, _VQD), lambda b: (b, 0, 0)),
            pl.BlockSpec((1, 1), lambda b: (0, 0)),
        ],
        out_shape=[
            jax.ShapeDtypeStruct((_B, _T2, _VQD), jnp.float32),
            jax.ShapeDtypeStruct((1, 1), jnp.float32),
        ],
    )(hmid, Wpost_t, b_post, Wout_t, bout2, codebook)

    loss = loss_sum[0, 0] / jnp.float32(_B * _T2 * _VQD)
    return (zq, loss, loss)
